# baseline (speedup 1.0000x reference)
import functools

import jax
import jax.numpy as jnp
from jax import lax
from jax.experimental import pallas as pl
from jax.experimental.pallas import tpu as pltpu
from jax.experimental.pallas import tpu_sc as plsc

H = 512
SEQ_LEN = 10
B = 16
N_SEG = B * SEQ_LEN
NSEGP = 256
T = 16384
TBLK = 512
N_HALF = 2
TH = T // N_HALF
NUM_TILES_H = TH // TBLK

SC_NC = 2
SC_NS = 16
SC_NW = SC_NC * SC_NS
ROWS_PER_W = TH // SC_NW
CHUNK = 64
NCH = ROWS_PER_W // CHUNK
SEG_PER_W = NSEGP // SC_NW

_SC_MESH = plsc.VectorSubcoreMesh(core_axis_name="c", subcore_axis_name="s")


def _em_gather_body(idx_h, ent_h, em_o, idx_v, rows0_v, rows1_v,
                    gsem0, gsem1, ssem0, ssem1):
    wid = lax.axis_index("s") * SC_NC + lax.axis_index("c")
    base_w = wid * ROWS_PER_W
    pltpu.sync_copy(idx_h.at[pl.ds(base_w, ROWS_PER_W)], idx_v)
    bufs = (rows0_v, rows1_v)
    gsems = (gsem0, gsem1)
    ssems = (ssem0, ssem1)

    def start_gather(c):
        b = c % 2
        return pltpu.async_copy(
            ent_h.at[idx_v.at[pl.ds(c * CHUNK, CHUNK)]], bufs[b], gsems[b])

    def start_store(c):
        b = c % 2
        return pltpu.async_copy(
            bufs[b], em_o.at[pl.ds(base_w + c * CHUNK, CHUNK)], ssems[b])

    hg = [None] * NCH
    hs = [None] * NCH
    hg[0] = start_gather(0)
    for c in range(NCH):
        if c + 1 < NCH:
            if c >= 1:
                hs[c - 1].wait()
            hg[c + 1] = start_gather(c + 1)
        hg[c].wait()
        hs[c] = start_store(c)
    hs[NCH - 2].wait()
    hs[NCH - 1].wait()


def _sc_gather_half(idx_half, ent_embeds):
    @functools.partial(
        pl.kernel,
        mesh=_SC_MESH,
        out_type=jax.ShapeDtypeStruct((TH, H), jnp.float32),
        scratch_types=[
            pltpu.VMEM((ROWS_PER_W,), jnp.int32),
            pltpu.VMEM((CHUNK, H), jnp.float32),
            pltpu.VMEM((CHUNK, H), jnp.float32),
            pltpu.SemaphoreType.DMA,
            pltpu.SemaphoreType.DMA,
            pltpu.SemaphoreType.DMA,
            pltpu.SemaphoreType.DMA,
        ],
    )
    def gather_k(idx_h, ent_h, em_o, idx_v, rows0_v, rows1_v,
                 gsem0, gsem1, ssem0, ssem1):
        _em_gather_body(idx_h, ent_h, em_o, idx_v, rows0_v, rows1_v,
                        gsem0, gsem1, ssem0, ssem1)

    return gather_k(idx_half, ent_embeds)


def _sc_gather_segs(s_idx_pad, r_idx_pad, ent_embeds, rel_embeds):
    @functools.partial(
        pl.kernel,
        mesh=_SC_MESH,
        out_type=(
            jax.ShapeDtypeStruct((NSEGP, H), jnp.float32),
            jax.ShapeDtypeStruct((NSEGP, H), jnp.float32),
        ),
        scratch_types=[
            pltpu.VMEM((SEG_PER_W,), jnp.int32),
            pltpu.VMEM((SEG_PER_W, H), jnp.float32),
            pltpu.VMEM((SEG_PER_W,), jnp.int32),
            pltpu.VMEM((SEG_PER_W, H), jnp.float32),
            pltpu.SemaphoreType.DMA,
            pltpu.SemaphoreType.DMA,
        ],
    )
    def gather_k(sidx_h, ridx_h, ent_h, rel_h, ss_o, rr_o,
                 idx_s, rows_s, idx_r, rows_r, sem_s, sem_r):
        wid = lax.axis_index("s") * SC_NC + lax.axis_index("c")
        sb = wid * SEG_PER_W
        pltpu.sync_copy(sidx_h.at[pl.ds(sb, SEG_PER_W)], idx_s)
        pltpu.sync_copy(ridx_h.at[pl.ds(sb, SEG_PER_W)], idx_r)
        h1 = pltpu.async_copy(ent_h.at[idx_s], rows_s, sem_s)
        h2 = pltpu.async_copy(rel_h.at[idx_r], rows_r, sem_r)
        h1.wait()
        pltpu.sync_copy(rows_s, ss_o.at[pl.ds(sb, SEG_PER_W)])
        h2.wait()
        pltpu.sync_copy(rows_r, rr_o.at[pl.ds(sb, SEG_PER_W)])

    return gather_k(s_idx_pad, r_idx_pad, ent_embeds, rel_embeds)


def _attn_tile(em_ref, seg_ref, w_ref, v_ref, bias_s, num_s, den_s):
    em = em_ref[...]
    seg = seg_ref[...]
    onehot = (seg == lax.broadcasted_iota(
        jnp.int32, (TBLK, NSEGP), 1)).astype(jnp.float32)
    ohb = onehot.astype(jnp.bfloat16)
    bias_tok = jnp.dot(ohb, bias_s[...].astype(jnp.bfloat16),
                       preferred_element_type=jnp.float32)
    z = jnp.dot(em.astype(jnp.bfloat16),
                w_ref[0:H, :].astype(jnp.bfloat16),
                preferred_element_type=jnp.float32) + bias_tok
    za = jnp.tanh(z)
    v = v_ref[...]
    s = jnp.sum(za * v, axis=1, keepdims=True)
    m_bound = jnp.sum(jnp.abs(v))
    e = jnp.exp(s - m_bound)
    num_s[...] += lax.dot_general(onehot, e * em, (((0,), (0,)), ((), ())),
                                  preferred_element_type=jnp.float32)
    den_s[...] += lax.dot_general(onehot, e, (((0,), (0,)), ((), ())),
                                  preferred_element_type=jnp.float32)


def _attn_body_first(em_ref, seg_ref, ss_ref, rr_ref, w_ref, b_ref, v_ref,
                     num_o, den_o, bias_o, bias_s, num_s, den_s):
    i = pl.program_id(0)

    @pl.when(i == 0)
    def _init():
        bias_s[...] = (
            jnp.dot(ss_ref[...], w_ref[H:2 * H, :],
                    preferred_element_type=jnp.float32)
            + jnp.dot(rr_ref[...], w_ref[2 * H:3 * H, :],
                      preferred_element_type=jnp.float32)
            + b_ref[...])
        num_s[...] = jnp.zeros_like(num_s)
        den_s[...] = jnp.zeros_like(den_s)

    _attn_tile(em_ref, seg_ref, w_ref, v_ref, bias_s, num_s, den_s)

    @pl.when(i == pl.num_programs(0) - 1)
    def _fin():
        num_o[...] = num_s[...]
        den_o[...] = den_s[...]
        bias_o[...] = bias_s[...]


def _attn_body_second(em_ref, seg_ref, ss_ref, rr_ref, w_ref, v_ref,
                      num_in, den_in, bias_in,
                      agg_o, ss_o, rr_o, bias_s, num_s, den_s):
    i = pl.program_id(0)

    @pl.when(i == 0)
    def _init():
        bias_s[...] = bias_in[...]
        num_s[...] = num_in[...]
        den_s[...] = den_in[...]

    _attn_tile(em_ref, seg_ref, w_ref, v_ref, bias_s, num_s, den_s)

    @pl.when(i == pl.num_programs(0) - 1)
    def _fin():
        den = den_s[...]
        mask = (den > 0).astype(jnp.float32)
        agg = num_s[...] / jnp.maximum(den, 1e-37)
        agg_o[...] = agg * mask
        ss_o[...] = ss_ref[...] * mask
        rr_o[...] = rr_ref[...] * mask


_FULL = lambda i: (0, 0)


def _tc_first(em0, seg0, ss_pad, rr_pad, W, b2, v2):
    return pl.pallas_call(
        _attn_body_first,
        grid=(NUM_TILES_H,),
        in_specs=[
            pl.BlockSpec((TBLK, H), lambda i: (i, 0)),
            pl.BlockSpec((TBLK, 1), lambda i: (i, 0)),
            pl.BlockSpec((NSEGP, H), _FULL),
            pl.BlockSpec((NSEGP, H), _FULL),
            pl.BlockSpec((3 * H, H), _FULL),
            pl.BlockSpec((1, H), _FULL),
            pl.BlockSpec((1, H), _FULL),
        ],
        out_specs=[
            pl.BlockSpec((NSEGP, H), _FULL),
            pl.BlockSpec((NSEGP, 1), _FULL),
            pl.BlockSpec((NSEGP, H), _FULL),
        ],
        out_shape=[
            jax.ShapeDtypeStruct((NSEGP, H), jnp.float32),
            jax.ShapeDtypeStruct((NSEGP, 1), jnp.float32),
            jax.ShapeDtypeStruct((NSEGP, H), jnp.float32),
        ],
        scratch_shapes=[
            pltpu.VMEM((NSEGP, H), jnp.float32),
            pltpu.VMEM((NSEGP, H), jnp.float32),
            pltpu.VMEM((NSEGP, 1), jnp.float32),
        ],
    )(em0, seg0, ss_pad, rr_pad, W, b2, v2)


def _tc_second(em1, seg1, ss_pad, rr_pad, W, v2, num_p, den_p, bias_p):
    return pl.pallas_call(
        _attn_body_second,
        grid=(NUM_TILES_H,),
        in_specs=[
            pl.BlockSpec((TBLK, H), lambda i: (i, 0)),
            pl.BlockSpec((TBLK, 1), lambda i: (i, 0)),
            pl.BlockSpec((NSEGP, H), _FULL),
            pl.BlockSpec((NSEGP, H), _FULL),
            pl.BlockSpec((3 * H, H), _FULL),
            pl.BlockSpec((1, H), _FULL),
            pl.BlockSpec((NSEGP, H), _FULL),
            pl.BlockSpec((NSEGP, 1), _FULL),
            pl.BlockSpec((NSEGP, H), _FULL),
        ],
        out_specs=[
            pl.BlockSpec((NSEGP, H), _FULL),
            pl.BlockSpec((NSEGP, H), _FULL),
            pl.BlockSpec((NSEGP, H), _FULL),
        ],
        out_shape=[
            jax.ShapeDtypeStruct((NSEGP, H), jnp.float32),
            jax.ShapeDtypeStruct((NSEGP, H), jnp.float32),
            jax.ShapeDtypeStruct((NSEGP, H), jnp.float32),
        ],
        scratch_shapes=[
            pltpu.VMEM((NSEGP, H), jnp.float32),
            pltpu.VMEM((NSEGP, H), jnp.float32),
            pltpu.VMEM((NSEGP, 1), jnp.float32),
        ],
    )(em1, seg1, ss_pad, rr_pad, W, v2, num_p, den_p, bias_p)


def kernel(flat_idx, segment_ids, s_idx, r_idx, ent_embeds, rel_embeds,
           W, b, v_s):
    flat_idx = flat_idx.astype(jnp.int32)
    s_idx_pad = jnp.pad(s_idx.astype(jnp.int32), (0, NSEGP - N_SEG))
    r_idx_pad = jnp.pad(r_idx.astype(jnp.int32), (0, NSEGP - N_SEG))

    ss_pad, rr_pad = _sc_gather_segs(s_idx_pad, r_idx_pad,
                                     ent_embeds, rel_embeds)
    em0 = _sc_gather_half(flat_idx[:TH], ent_embeds)
    em1 = _sc_gather_half(flat_idx[TH:], ent_embeds)

    seg_col = segment_ids.astype(jnp.int32).reshape(T, 1)
    b2 = b.reshape(1, H)
    v2 = v_s.reshape(1, H)

    num_p, den_p, bias_p = _tc_first(em0, seg_col[:TH], ss_pad, rr_pad,
                                     W, b2, v2)
    agg, ssm, rrm = _tc_second(em1, seg_col[TH:], ss_pad, rr_pad, W, v2,
                               num_p, den_p, bias_p)

    row = jnp.concatenate([agg, ssm, rrm], axis=1)[:N_SEG]
    return row.reshape(B, SEQ_LEN, 3 * H)


# speedup vs baseline: 1.1406x; 1.1406x over previous
import functools

import jax
import jax.numpy as jnp
from jax import lax
from jax.experimental import pallas as pl
from jax.experimental.pallas import tpu as pltpu
from jax.experimental.pallas import tpu_sc as plsc

H = 512
SEQ_LEN = 10
B = 16
NSEG = B * SEQ_LEN
T = 16384
TBLK = 512
N_HALF = 2
TH = T // N_HALF
NUM_TILES_H = TH // TBLK

SC_NC = 2
SC_NS = 16
SC_NW = SC_NC * SC_NS
ROWS_PER_W = TH // SC_NW
CHUNK = 64
NCH = ROWS_PER_W // CHUNK
SEG_PER_W = 8

def _sc_mesh():
    return plsc.VectorSubcoreMesh(core_axis_name="c", subcore_axis_name="s")


def _em_gather_chunks(idx_h, ent_h, em_o, idx_v, rows0_v, rows1_v,
                      gsem0, gsem1, ssem0, ssem1, idx_off, wid):
    base_w = wid * ROWS_PER_W
    pltpu.sync_copy(idx_h.at[pl.ds(idx_off + base_w, ROWS_PER_W)], idx_v)
    bufs = (rows0_v, rows1_v)
    gsems = (gsem0, gsem1)
    ssems = (ssem0, ssem1)

    def start_gather(c):
        b = c % 2
        return pltpu.async_copy(
            ent_h.at[idx_v.at[pl.ds(c * CHUNK, CHUNK)]], bufs[b], gsems[b])

    def start_store(c):
        b = c % 2
        return pltpu.async_copy(
            bufs[b], em_o.at[pl.ds(base_w + c * CHUNK, CHUNK)], ssems[b])

    hg = [None] * NCH
    hs = [None] * NCH
    hg[0] = start_gather(0)
    for c in range(NCH):
        if c + 1 < NCH:
            if c >= 1:
                hs[c - 1].wait()
            hg[c + 1] = start_gather(c + 1)
        hg[c].wait()
        hs[c] = start_store(c)
    hs[NCH - 2].wait()
    hs[NCH - 1].wait()


def _sc_gather_first(flat_idx, s_idx, r_idx, ent_embeds, rel_embeds):
    @functools.partial(
        pl.kernel,
        mesh=_sc_mesh(),
        out_type=(
            jax.ShapeDtypeStruct((TH, H), jnp.float32),
            jax.ShapeDtypeStruct((NSEG, H), jnp.float32),
            jax.ShapeDtypeStruct((NSEG, H), jnp.float32),
        ),
        scratch_types=[
            pltpu.VMEM((ROWS_PER_W,), jnp.int32),
            pltpu.VMEM((CHUNK, H), jnp.float32),
            pltpu.VMEM((CHUNK, H), jnp.float32),
            pltpu.VMEM((SEG_PER_W,), jnp.int32),
            pltpu.VMEM((SEG_PER_W, H), jnp.float32),
            pltpu.SemaphoreType.DMA,
            pltpu.SemaphoreType.DMA,
            pltpu.SemaphoreType.DMA,
            pltpu.SemaphoreType.DMA,
        ],
    )
    def gather_k(idx_h, sidx_h, ridx_h, ent_h, rel_h, em_o, ss_o, rr_o,
                 idx_v, rows0_v, rows1_v, idx_s, rows_s,
                 gsem0, gsem1, ssem0, ssem1):
        wid = lax.axis_index("s") * SC_NC + lax.axis_index("c")

        @pl.when(wid < NSEG // SEG_PER_W)
        def _segs():
            sb = wid * SEG_PER_W
            pltpu.sync_copy(sidx_h.at[pl.ds(sb, SEG_PER_W)], idx_s)
            pltpu.async_copy(ent_h.at[idx_s], rows_s, gsem0).wait()
            pltpu.sync_copy(rows_s, ss_o.at[pl.ds(sb, SEG_PER_W)])
            pltpu.sync_copy(ridx_h.at[pl.ds(sb, SEG_PER_W)], idx_s)
            pltpu.async_copy(rel_h.at[idx_s], rows_s, gsem0).wait()
            pltpu.sync_copy(rows_s, rr_o.at[pl.ds(sb, SEG_PER_W)])

        _em_gather_chunks(idx_h, ent_h, em_o, idx_v, rows0_v, rows1_v,
                          gsem0, gsem1, ssem0, ssem1, 0, wid)

    return gather_k(flat_idx, s_idx, r_idx, ent_embeds, rel_embeds)


def _sc_gather_second(flat_idx, ent_embeds):
    @functools.partial(
        pl.kernel,
        mesh=_sc_mesh(),
        out_type=jax.ShapeDtypeStruct((TH, H), jnp.float32),
        scratch_types=[
            pltpu.VMEM((ROWS_PER_W,), jnp.int32),
            pltpu.VMEM((CHUNK, H), jnp.float32),
            pltpu.VMEM((CHUNK, H), jnp.float32),
            pltpu.SemaphoreType.DMA,
            pltpu.SemaphoreType.DMA,
            pltpu.SemaphoreType.DMA,
            pltpu.SemaphoreType.DMA,
        ],
    )
    def gather_k(idx_h, ent_h, em_o, idx_v, rows0_v, rows1_v,
                 gsem0, gsem1, ssem0, ssem1):
        wid = lax.axis_index("s") * SC_NC + lax.axis_index("c")
        _em_gather_chunks(idx_h, ent_h, em_o, idx_v, rows0_v, rows1_v,
                          gsem0, gsem1, ssem0, ssem1, TH, wid)

    return gather_k(flat_idx, ent_embeds)


def _attn_tile(em_ref, seg_ref, w_ref, v_ref, bias_s, num_s, den_s):
    em = em_ref[...]
    seg = seg_ref[...]
    onehot = (seg == lax.broadcasted_iota(
        jnp.int32, (TBLK, NSEG), 1)).astype(jnp.float32)
    ohb = onehot.astype(jnp.bfloat16)
    bias_tok = jnp.dot(ohb, bias_s[...].astype(jnp.bfloat16),
                       preferred_element_type=jnp.float32)
    z = jnp.dot(em.astype(jnp.bfloat16),
                w_ref[0:H, :].astype(jnp.bfloat16),
                preferred_element_type=jnp.float32) + bias_tok
    za = jnp.tanh(z)
    v = v_ref[...]
    s = jnp.sum(za * v, axis=1, keepdims=True)
    m_bound = jnp.sum(jnp.abs(v))
    e = jnp.exp(s - m_bound)
    ohe = onehot * e
    num_s[...] += lax.dot_general(ohe, em, (((0,), (0,)), ((), ())),
                                  preferred_element_type=jnp.float32)
    den_s[...] += jnp.sum(ohe, axis=0, keepdims=True)


def _attn_body_first(em_ref, seg_ref, ss_ref, rr_ref, w_ref, b_ref, v_ref,
                     num_o, den_o, bias_o, bias_s, num_s, den_s):
    i = pl.program_id(0)

    @pl.when(i == 0)
    def _init():
        bias_s[...] = (
            jnp.dot(ss_ref[...], w_ref[H:2 * H, :],
                    preferred_element_type=jnp.float32)
            + jnp.dot(rr_ref[...], w_ref[2 * H:3 * H, :],
                      preferred_element_type=jnp.float32)
            + b_ref[...])
        num_s[...] = jnp.zeros_like(num_s)
        den_s[...] = jnp.zeros_like(den_s)

    _attn_tile(em_ref, seg_ref, w_ref, v_ref, bias_s, num_s, den_s)

    @pl.when(i == pl.num_programs(0) - 1)
    def _fin():
        num_o[...] = num_s[...]
        den_o[...] = den_s[...]
        bias_o[...] = bias_s[...]


def _attn_body_second(em_ref, seg_ref, ss_ref, rr_ref, w_ref, v_ref,
                      num_in, den_in, bias_in,
                      row_o, bias_s, num_s, den_s):
    i = pl.program_id(0)

    @pl.when(i == 0)
    def _init():
        bias_s[...] = bias_in[...]
        num_s[...] = num_in[...]
        den_s[...] = den_in[...]

    _attn_tile(em_ref, seg_ref, w_ref, v_ref, bias_s, num_s, den_s)

    @pl.when(i == pl.num_programs(0) - 1)
    def _fin():
        den = jnp.transpose(den_s[...])
        mask = (den > 0).astype(jnp.float32)
        agg = num_s[...] / jnp.maximum(den, 1e-37)
        row_o[:, 0:H] = agg * mask
        row_o[:, H:2 * H] = ss_ref[...] * mask
        row_o[:, 2 * H:3 * H] = rr_ref[...] * mask


_FULL = lambda i: (0, 0)


def _tc_first(em0, seg_col, ss, rr, W, b2, v2):
    return pl.pallas_call(
        _attn_body_first,
        grid=(NUM_TILES_H,),
        in_specs=[
            pl.BlockSpec((TBLK, H), lambda i: (i, 0)),
            pl.BlockSpec((TBLK, 1), lambda i: (i, 0)),
            pl.BlockSpec((NSEG, H), _FULL),
            pl.BlockSpec((NSEG, H), _FULL),
            pl.BlockSpec((3 * H, H), _FULL),
            pl.BlockSpec((1, H), _FULL),
            pl.BlockSpec((1, H), _FULL),
        ],
        out_specs=[
            pl.BlockSpec((NSEG, H), _FULL),
            pl.BlockSpec((1, NSEG), _FULL),
            pl.BlockSpec((NSEG, H), _FULL),
        ],
        out_shape=[
            jax.ShapeDtypeStruct((NSEG, H), jnp.float32),
            jax.ShapeDtypeStruct((1, NSEG), jnp.float32),
            jax.ShapeDtypeStruct((NSEG, H), jnp.float32),
        ],
        scratch_shapes=[
            pltpu.VMEM((NSEG, H), jnp.float32),
            pltpu.VMEM((NSEG, H), jnp.float32),
            pltpu.VMEM((1, NSEG), jnp.float32),
        ],
    )(em0, seg_col, ss, rr, W, b2, v2)


def _tc_second(em1, seg_col, ss, rr, W, v2, num_p, den_p, bias_p):
    return pl.pallas_call(
        _attn_body_second,
        grid=(NUM_TILES_H,),
        in_specs=[
            pl.BlockSpec((TBLK, H), lambda i: (i, 0)),
            pl.BlockSpec((TBLK, 1), lambda i: (i + NUM_TILES_H, 0)),
            pl.BlockSpec((NSEG, H), _FULL),
            pl.BlockSpec((NSEG, H), _FULL),
            pl.BlockSpec((3 * H, H), _FULL),
            pl.BlockSpec((1, H), _FULL),
            pl.BlockSpec((NSEG, H), _FULL),
            pl.BlockSpec((1, NSEG), _FULL),
            pl.BlockSpec((NSEG, H), _FULL),
        ],
        out_specs=pl.BlockSpec((NSEG, 3 * H), _FULL),
        out_shape=jax.ShapeDtypeStruct((NSEG, 3 * H), jnp.float32),
        scratch_shapes=[
            pltpu.VMEM((NSEG, H), jnp.float32),
            pltpu.VMEM((NSEG, H), jnp.float32),
            pltpu.VMEM((1, NSEG), jnp.float32),
        ],
    )(em1, seg_col, ss, rr, W, v2, num_p, den_p, bias_p)


def kernel(flat_idx, segment_ids, s_idx, r_idx, ent_embeds, rel_embeds,
           W, b, v_s):
    flat_idx = flat_idx.astype(jnp.int32)
    s_idx = s_idx.astype(jnp.int32)
    r_idx = r_idx.astype(jnp.int32)

    em0, ss, rr = _sc_gather_first(flat_idx, s_idx, r_idx,
                                   ent_embeds, rel_embeds)
    em1 = _sc_gather_second(flat_idx, ent_embeds)

    seg_col = segment_ids.astype(jnp.int32).reshape(T, 1)
    b2 = b.reshape(1, H)
    v2 = v_s.reshape(1, H)

    num_p, den_p, bias_p = _tc_first(em0, seg_col, ss, rr, W, b2, v2)
    row = _tc_second(em1, seg_col, ss, rr, W, v2, num_p, den_p, bias_p)
    return row.reshape(B, SEQ_LEN, 3 * H)


# speedup vs baseline: 1.2037x; 1.0553x over previous
import functools

import jax
import jax.numpy as jnp
from jax import lax
from jax.experimental import pallas as pl
from jax.experimental.pallas import tpu as pltpu
from jax.experimental.pallas import tpu_sc as plsc

H = 512
SEQ_LEN = 10
B = 16
NSEG = B * SEQ_LEN
T = 16384
TBLK = 512
N_HALF = 2
TH = T // N_HALF
NUM_TILES_H = TH // TBLK

SC_NC = 2
SC_NS = 16
SC_NW = SC_NC * SC_NS
ROWS_PER_W = TH // SC_NW
CHUNK = 64
NCH = ROWS_PER_W // CHUNK
SEG_PER_W = 8

def _sc_mesh():
    return plsc.VectorSubcoreMesh(core_axis_name="c", subcore_axis_name="s")


def _em_gather_chunks(idx_h, ent_h, em_o, idx_v, rows0_v, rows1_v,
                      gsem0, gsem1, ssem0, ssem1, idx_off, wid):
    base_w = wid * ROWS_PER_W
    pltpu.sync_copy(idx_h.at[pl.ds(idx_off + base_w, ROWS_PER_W)], idx_v)
    bufs = (rows0_v, rows1_v)
    gsems = (gsem0, gsem1)
    ssems = (ssem0, ssem1)

    def start_gather(c):
        b = c % 2
        return pltpu.async_copy(
            ent_h.at[idx_v.at[pl.ds(c * CHUNK, CHUNK)]], bufs[b], gsems[b])

    def start_store(c):
        b = c % 2
        return pltpu.async_copy(
            bufs[b], em_o.at[pl.ds(base_w + c * CHUNK, CHUNK)], ssems[b])

    hg = [None] * NCH
    hs = [None] * NCH
    hg[0] = start_gather(0)
    for c in range(NCH):
        if c + 1 < NCH:
            if c >= 1:
                hs[c - 1].wait()
            hg[c + 1] = start_gather(c + 1)
        hg[c].wait()
        hs[c] = start_store(c)
    hs[NCH - 2].wait()
    hs[NCH - 1].wait()


def _sc_gather_first(flat_idx, s_idx, r_idx, ent_embeds, rel_embeds):
    @functools.partial(
        pl.kernel,
        mesh=_sc_mesh(),
        out_type=(
            jax.ShapeDtypeStruct((TH, H), jnp.float32),
            jax.ShapeDtypeStruct((NSEG, H), jnp.float32),
            jax.ShapeDtypeStruct((NSEG, H), jnp.float32),
        ),
        scratch_types=[
            pltpu.VMEM((ROWS_PER_W,), jnp.int32),
            pltpu.VMEM((CHUNK, H), jnp.float32),
            pltpu.VMEM((CHUNK, H), jnp.float32),
            pltpu.VMEM((SEG_PER_W,), jnp.int32),
            pltpu.VMEM((SEG_PER_W,), jnp.int32),
            pltpu.VMEM((SEG_PER_W, H), jnp.float32),
            pltpu.VMEM((SEG_PER_W, H), jnp.float32),
            pltpu.SemaphoreType.DMA,
            pltpu.SemaphoreType.DMA,
            pltpu.SemaphoreType.DMA,
            pltpu.SemaphoreType.DMA,
            pltpu.SemaphoreType.DMA,
            pltpu.SemaphoreType.DMA,
        ],
    )
    def gather_k(idx_h, sidx_h, ridx_h, ent_h, rel_h, em_o, ss_o, rr_o,
                 idx_v, rows0_v, rows1_v, idx_s, idx_r, rows_s, rows_r,
                 gsem0, gsem1, ssem0, ssem1, sem_s, sem_r):
        wid = lax.axis_index("s") * SC_NC + lax.axis_index("c")
        is_seg = wid < NSEG // SEG_PER_W
        sb = wid * SEG_PER_W

        @pl.when(is_seg)
        def _seg_start():
            pltpu.sync_copy(sidx_h.at[pl.ds(sb, SEG_PER_W)], idx_s)
            pltpu.sync_copy(ridx_h.at[pl.ds(sb, SEG_PER_W)], idx_r)
            pltpu.async_copy(ent_h.at[idx_s], rows_s, sem_s)
            pltpu.async_copy(rel_h.at[idx_r], rows_r, sem_r)

        _em_gather_chunks(idx_h, ent_h, em_o, idx_v, rows0_v, rows1_v,
                          gsem0, gsem1, ssem0, ssem1, 0, wid)

        @pl.when(is_seg)
        def _seg_drain():
            pltpu.make_async_copy(ent_h.at[idx_s], rows_s, sem_s).wait()
            pltpu.sync_copy(rows_s, ss_o.at[pl.ds(sb, SEG_PER_W)])
            pltpu.make_async_copy(rel_h.at[idx_r], rows_r, sem_r).wait()
            pltpu.sync_copy(rows_r, rr_o.at[pl.ds(sb, SEG_PER_W)])

    return gather_k(flat_idx, s_idx, r_idx, ent_embeds, rel_embeds)


def _sc_gather_second(flat_idx, ent_embeds):
    @functools.partial(
        pl.kernel,
        mesh=_sc_mesh(),
        out_type=jax.ShapeDtypeStruct((TH, H), jnp.float32),
        scratch_types=[
            pltpu.VMEM((ROWS_PER_W,), jnp.int32),
            pltpu.VMEM((CHUNK, H), jnp.float32),
            pltpu.VMEM((CHUNK, H), jnp.float32),
            pltpu.SemaphoreType.DMA,
            pltpu.SemaphoreType.DMA,
            pltpu.SemaphoreType.DMA,
            pltpu.SemaphoreType.DMA,
        ],
    )
    def gather_k(idx_h, ent_h, em_o, idx_v, rows0_v, rows1_v,
                 gsem0, gsem1, ssem0, ssem1):
        wid = lax.axis_index("s") * SC_NC + lax.axis_index("c")
        _em_gather_chunks(idx_h, ent_h, em_o, idx_v, rows0_v, rows1_v,
                          gsem0, gsem1, ssem0, ssem1, TH, wid)

    return gather_k(flat_idx, ent_embeds)


def _attn_tile(em_ref, seg_ref, w_ref, v_ref, bias_s, num_s, den_s):
    em = em_ref[...]
    seg = seg_ref[...]
    onehot_t = (seg[None, :] == lax.broadcasted_iota(
        jnp.int32, (NSEG, TBLK), 0)).astype(jnp.float32)
    ohb = onehot_t.astype(jnp.bfloat16)
    bias_t = lax.dot_general(bias_s[...].astype(jnp.bfloat16), ohb,
                             (((0,), (0,)), ((), ())),
                             preferred_element_type=jnp.float32)
    zt = lax.dot_general(w_ref[0:H, :].astype(jnp.bfloat16),
                         em.astype(jnp.bfloat16),
                         (((0,), (1,)), ((), ())),
                         preferred_element_type=jnp.float32) + bias_t
    zat = jnp.tanh(zt)
    v = v_ref[...]
    s_row = jnp.sum(zat * v, axis=0, keepdims=True)
    m_bound = jnp.sum(jnp.abs(v))
    e_row = jnp.exp(s_row - m_bound)
    ohe = onehot_t * e_row
    num_s[...] += lax.dot_general(ohe, em, (((1,), (0,)), ((), ())),
                                  preferred_element_type=jnp.float32)
    den_s[...] += jnp.sum(ohe, axis=1, keepdims=True)


def _attn_body_first(em_ref, seg_ref, ss_ref, rr_ref, w_ref, b_ref, v_ref,
                     num_o, den_o, bias_o, bias_s, num_s, den_s):
    i = pl.program_id(0)

    @pl.when(i == 0)
    def _init():
        bias_s[...] = (
            jnp.dot(ss_ref[...].astype(jnp.bfloat16),
                    w_ref[H:2 * H, :].astype(jnp.bfloat16),
                    preferred_element_type=jnp.float32)
            + jnp.dot(rr_ref[...].astype(jnp.bfloat16),
                      w_ref[2 * H:3 * H, :].astype(jnp.bfloat16),
                      preferred_element_type=jnp.float32)
            + b_ref[...])
        num_s[...] = jnp.zeros_like(num_s)
        den_s[...] = jnp.zeros_like(den_s)

    _attn_tile(em_ref, seg_ref, w_ref, v_ref, bias_s, num_s, den_s)

    @pl.when(i == pl.num_programs(0) - 1)
    def _fin():
        num_o[...] = num_s[...]
        den_o[...] = den_s[...]
        bias_o[...] = bias_s[...]


def _attn_body_second(em_ref, seg_ref, ss_ref, rr_ref, w_ref, v_ref,
                      num_in, den_in, bias_in,
                      row_o, bias_s, num_s, den_s):
    i = pl.program_id(0)

    @pl.when(i == 0)
    def _init():
        bias_s[...] = bias_in[...]
        num_s[...] = num_in[...]
        den_s[...] = den_in[...]

    _attn_tile(em_ref, seg_ref, w_ref, v_ref, bias_s, num_s, den_s)

    @pl.when(i == pl.num_programs(0) - 1)
    def _fin():
        den = den_s[...]
        mask = (den > 0).astype(jnp.float32)
        agg = num_s[...] / jnp.maximum(den, 1e-37)
        row_o[:, 0:H] = agg * mask
        row_o[:, H:2 * H] = ss_ref[...] * mask
        row_o[:, 2 * H:3 * H] = rr_ref[...] * mask


_FULL = lambda i: (0, 0)


def _tc_first(em0, seg_col, ss, rr, W, b2, v2):
    return pl.pallas_call(
        _attn_body_first,
        grid=(NUM_TILES_H,),
        in_specs=[
            pl.BlockSpec((TBLK, H), lambda i: (i, 0)),
            pl.BlockSpec((TBLK,), lambda i: (i,)),
            pl.BlockSpec((NSEG, H), _FULL),
            pl.BlockSpec((NSEG, H), _FULL),
            pl.BlockSpec((3 * H, H), _FULL),
            pl.BlockSpec((1, H), _FULL),
            pl.BlockSpec((H, 1), _FULL),
        ],
        out_specs=[
            pl.BlockSpec((NSEG, H), _FULL),
            pl.BlockSpec((NSEG, 1), _FULL),
            pl.BlockSpec((NSEG, H), _FULL),
        ],
        out_shape=[
            jax.ShapeDtypeStruct((NSEG, H), jnp.float32),
            jax.ShapeDtypeStruct((NSEG, 1), jnp.float32),
            jax.ShapeDtypeStruct((NSEG, H), jnp.float32),
        ],
        scratch_shapes=[
            pltpu.VMEM((NSEG, H), jnp.float32),
            pltpu.VMEM((NSEG, H), jnp.float32),
            pltpu.VMEM((NSEG, 1), jnp.float32),
        ],
    )(em0, seg_col, ss, rr, W, b2, v2)


def _tc_second(em1, seg_col, ss, rr, W, v2, num_p, den_p, bias_p):
    return pl.pallas_call(
        _attn_body_second,
        grid=(NUM_TILES_H,),
        in_specs=[
            pl.BlockSpec((TBLK, H), lambda i: (i, 0)),
            pl.BlockSpec((TBLK,), lambda i: (i + NUM_TILES_H,)),
            pl.BlockSpec((NSEG, H), _FULL),
            pl.BlockSpec((NSEG, H), _FULL),
            pl.BlockSpec((3 * H, H), _FULL),
            pl.BlockSpec((H, 1), _FULL),
            pl.BlockSpec((NSEG, H), _FULL),
            pl.BlockSpec((NSEG, 1), _FULL),
            pl.BlockSpec((NSEG, H), _FULL),
        ],
        out_specs=pl.BlockSpec((NSEG, 3 * H), _FULL),
        out_shape=jax.ShapeDtypeStruct((NSEG, 3 * H), jnp.float32),
        scratch_shapes=[
            pltpu.VMEM((NSEG, H), jnp.float32),
            pltpu.VMEM((NSEG, H), jnp.float32),
            pltpu.VMEM((NSEG, 1), jnp.float32),
        ],
    )(em1, seg_col, ss, rr, W, v2, num_p, den_p, bias_p)


def kernel(flat_idx, segment_ids, s_idx, r_idx, ent_embeds, rel_embeds,
           W, b, v_s):
    flat_idx = flat_idx.astype(jnp.int32)
    s_idx = s_idx.astype(jnp.int32)
    r_idx = r_idx.astype(jnp.int32)

    em0, ss, rr = _sc_gather_first(flat_idx, s_idx, r_idx,
                                   ent_embeds, rel_embeds)
    em1 = _sc_gather_second(flat_idx, ent_embeds)

    seg_ids = segment_ids.astype(jnp.int32)
    b2 = b.reshape(1, H)

    num_p, den_p, bias_p = _tc_first(em0, seg_ids, ss, rr, W, b2, v_s)
    row = _tc_second(em1, seg_ids, ss, rr, W, v_s, num_p, den_p, bias_p)
    return row.reshape(B, SEQ_LEN, 3 * H)


# speedup vs baseline: 1.3803x; 1.1468x over previous
import functools

import jax
import jax.numpy as jnp
from jax import lax
from jax.experimental import pallas as pl
from jax.experimental.pallas import tpu as pltpu
from jax.experimental.pallas import tpu_sc as plsc

H = 512
SEQ_LEN = 10
B = 16
NSEG = B * SEQ_LEN
T = 16384
TBLK = 1024
N_HALF = 2
TH = T // N_HALF
NUM_TILES_H = TH // TBLK

SC_NC = 2
SC_NS = 16
SC_NW = SC_NC * SC_NS
ROWS_PER_W = TH // SC_NW
CHUNK = 64
NCH = ROWS_PER_W // CHUNK
SEG_PER_W = 8

def _sc_mesh():
    return plsc.VectorSubcoreMesh(core_axis_name="c", subcore_axis_name="s")


def _em_gather_chunks(idx_h, ent_h, em_o, idx_v, rows0_v, rows1_v,
                      gsem0, gsem1, ssem0, ssem1, idx_off, wid):
    base_w = wid * ROWS_PER_W
    pltpu.sync_copy(idx_h.at[pl.ds(idx_off + base_w, ROWS_PER_W)], idx_v)
    bufs = (rows0_v, rows1_v)
    gsems = (gsem0, gsem1)
    ssems = (ssem0, ssem1)

    def start_gather(c):
        b = c % 2
        return pltpu.async_copy(
            ent_h.at[idx_v.at[pl.ds(c * CHUNK, CHUNK)]], bufs[b], gsems[b])

    def start_store(c):
        b = c % 2
        return pltpu.async_copy(
            bufs[b], em_o.at[pl.ds(base_w + c * CHUNK, CHUNK)], ssems[b])

    hg = [None] * NCH
    hs = [None] * NCH
    hg[0] = start_gather(0)
    for c in range(NCH):
        if c + 1 < NCH:
            if c >= 1:
                hs[c - 1].wait()
            hg[c + 1] = start_gather(c + 1)
        hg[c].wait()
        hs[c] = start_store(c)
    hs[NCH - 2].wait()
    hs[NCH - 1].wait()


def _sc_gather_first(flat_idx, s_idx, r_idx, ent_embeds, rel_embeds):
    @functools.partial(
        pl.kernel,
        mesh=_sc_mesh(),
        out_type=(
            jax.ShapeDtypeStruct((TH, H), jnp.float32),
            jax.ShapeDtypeStruct((NSEG, H), jnp.float32),
            jax.ShapeDtypeStruct((NSEG, H), jnp.float32),
        ),
        scratch_types=[
            pltpu.VMEM((ROWS_PER_W,), jnp.int32),
            pltpu.VMEM((CHUNK, H), jnp.float32),
            pltpu.VMEM((CHUNK, H), jnp.float32),
            pltpu.VMEM((SEG_PER_W,), jnp.int32),
            pltpu.VMEM((SEG_PER_W,), jnp.int32),
            pltpu.VMEM((SEG_PER_W, H), jnp.float32),
            pltpu.VMEM((SEG_PER_W, H), jnp.float32),
            pltpu.SemaphoreType.DMA,
            pltpu.SemaphoreType.DMA,
            pltpu.SemaphoreType.DMA,
            pltpu.SemaphoreType.DMA,
            pltpu.SemaphoreType.DMA,
            pltpu.SemaphoreType.DMA,
        ],
    )
    def gather_k(idx_h, sidx_h, ridx_h, ent_h, rel_h, em_o, ss_o, rr_o,
                 idx_v, rows0_v, rows1_v, idx_s, idx_r, rows_s, rows_r,
                 gsem0, gsem1, ssem0, ssem1, sem_s, sem_r):
        wid = lax.axis_index("s") * SC_NC + lax.axis_index("c")
        is_seg = wid < NSEG // SEG_PER_W
        sb = wid * SEG_PER_W

        @pl.when(is_seg)
        def _seg_start():
            pltpu.sync_copy(sidx_h.at[pl.ds(sb, SEG_PER_W)], idx_s)
            pltpu.sync_copy(ridx_h.at[pl.ds(sb, SEG_PER_W)], idx_r)
            pltpu.async_copy(ent_h.at[idx_s], rows_s, sem_s)
            pltpu.async_copy(rel_h.at[idx_r], rows_r, sem_r)

        _em_gather_chunks(idx_h, ent_h, em_o, idx_v, rows0_v, rows1_v,
                          gsem0, gsem1, ssem0, ssem1, 0, wid)

        @pl.when(is_seg)
        def _seg_drain():
            pltpu.make_async_copy(ent_h.at[idx_s], rows_s, sem_s).wait()
            pltpu.sync_copy(rows_s, ss_o.at[pl.ds(sb, SEG_PER_W)])
            pltpu.make_async_copy(rel_h.at[idx_r], rows_r, sem_r).wait()
            pltpu.sync_copy(rows_r, rr_o.at[pl.ds(sb, SEG_PER_W)])

    return gather_k(flat_idx, s_idx, r_idx, ent_embeds, rel_embeds)


def _sc_gather_second(flat_idx, ent_embeds):
    @functools.partial(
        pl.kernel,
        mesh=_sc_mesh(),
        out_type=jax.ShapeDtypeStruct((TH, H), jnp.float32),
        scratch_types=[
            pltpu.VMEM((ROWS_PER_W,), jnp.int32),
            pltpu.VMEM((CHUNK, H), jnp.float32),
            pltpu.VMEM((CHUNK, H), jnp.float32),
            pltpu.SemaphoreType.DMA,
            pltpu.SemaphoreType.DMA,
            pltpu.SemaphoreType.DMA,
            pltpu.SemaphoreType.DMA,
        ],
    )
    def gather_k(idx_h, ent_h, em_o, idx_v, rows0_v, rows1_v,
                 gsem0, gsem1, ssem0, ssem1):
        wid = lax.axis_index("s") * SC_NC + lax.axis_index("c")
        _em_gather_chunks(idx_h, ent_h, em_o, idx_v, rows0_v, rows1_v,
                          gsem0, gsem1, ssem0, ssem1, TH, wid)

    return gather_k(flat_idx, ent_embeds)


def _attn_tile(em_ref, seg_ref, w_ref, v_ref, bias_s, num_s, den_s):
    em = em_ref[...]
    seg = seg_ref[...]
    onehot_t = (seg[None, :] == lax.broadcasted_iota(
        jnp.int32, (NSEG, TBLK), 0)).astype(jnp.float32)
    ohb = onehot_t.astype(jnp.bfloat16)
    bias_t = lax.dot_general(bias_s[...].astype(jnp.bfloat16), ohb,
                             (((0,), (0,)), ((), ())),
                             preferred_element_type=jnp.float32)
    zt = lax.dot_general(w_ref[0:H, :].astype(jnp.bfloat16),
                         em.astype(jnp.bfloat16),
                         (((0,), (1,)), ((), ())),
                         preferred_element_type=jnp.float32) + bias_t
    zat = jnp.tanh(zt)
    v = v_ref[...]
    s_row = jnp.sum(zat * v, axis=0, keepdims=True)
    m_bound = jnp.sum(jnp.abs(v))
    e_row = jnp.exp(s_row - m_bound)
    ohe = onehot_t * e_row
    num_s[...] += lax.dot_general(ohe, em, (((1,), (0,)), ((), ())),
                                  preferred_element_type=jnp.float32)
    den_s[...] += jnp.sum(ohe, axis=1, keepdims=True)


def _attn_body_first(em_ref, seg_ref, ss_ref, rr_ref, w_ref, b_ref, v_ref,
                     num_o, den_o, bias_o, bias_s, num_s, den_s):
    i = pl.program_id(0)

    @pl.when(i == 0)
    def _init():
        bias_s[...] = (
            jnp.dot(ss_ref[...].astype(jnp.bfloat16),
                    w_ref[H:2 * H, :].astype(jnp.bfloat16),
                    preferred_element_type=jnp.float32)
            + jnp.dot(rr_ref[...].astype(jnp.bfloat16),
                      w_ref[2 * H:3 * H, :].astype(jnp.bfloat16),
                      preferred_element_type=jnp.float32)
            + b_ref[...])
        num_s[...] = jnp.zeros_like(num_s)
        den_s[...] = jnp.zeros_like(den_s)

    _attn_tile(em_ref, seg_ref, w_ref, v_ref, bias_s, num_s, den_s)

    @pl.when(i == pl.num_programs(0) - 1)
    def _fin():
        num_o[...] = num_s[...]
        den_o[...] = den_s[...]
        bias_o[...] = bias_s[...]


def _attn_body_second(em_ref, seg_ref, ss_ref, rr_ref, w_ref, v_ref,
                      num_in, den_in, bias_in,
                      row_o, bias_s, num_s, den_s):
    i = pl.program_id(0)

    @pl.when(i == 0)
    def _init():
        bias_s[...] = bias_in[...]
        num_s[...] = num_in[...]
        den_s[...] = den_in[...]

    _attn_tile(em_ref, seg_ref, w_ref, v_ref, bias_s, num_s, den_s)

    @pl.when(i == pl.num_programs(0) - 1)
    def _fin():
        den = den_s[...]
        mask = (den > 0).astype(jnp.float32)
        agg = num_s[...] / jnp.maximum(den, 1e-37)
        row_o[:, 0:H] = agg * mask
        row_o[:, H:2 * H] = ss_ref[...] * mask
        row_o[:, 2 * H:3 * H] = rr_ref[...] * mask


_FULL = lambda i: (0, 0)


def _tc_first(em0, seg_col, ss, rr, W, b2, v2):
    return pl.pallas_call(
        _attn_body_first,
        grid=(NUM_TILES_H,),
        in_specs=[
            pl.BlockSpec((TBLK, H), lambda i: (i, 0)),
            pl.BlockSpec((TBLK,), lambda i: (i,)),
            pl.BlockSpec((NSEG, H), _FULL),
            pl.BlockSpec((NSEG, H), _FULL),
            pl.BlockSpec((3 * H, H), _FULL),
            pl.BlockSpec((1, H), _FULL),
            pl.BlockSpec((H, 1), _FULL),
        ],
        out_specs=[
            pl.BlockSpec((NSEG, H), _FULL),
            pl.BlockSpec((NSEG, 1), _FULL),
            pl.BlockSpec((NSEG, H), _FULL),
        ],
        out_shape=[
            jax.ShapeDtypeStruct((NSEG, H), jnp.float32),
            jax.ShapeDtypeStruct((NSEG, 1), jnp.float32),
            jax.ShapeDtypeStruct((NSEG, H), jnp.float32),
        ],
        scratch_shapes=[
            pltpu.VMEM((NSEG, H), jnp.float32),
            pltpu.VMEM((NSEG, H), jnp.float32),
            pltpu.VMEM((NSEG, 1), jnp.float32),
        ],
    )(em0, seg_col, ss, rr, W, b2, v2)


def _tc_second(em1, seg_col, ss, rr, W, v2, num_p, den_p, bias_p):
    return pl.pallas_call(
        _attn_body_second,
        grid=(NUM_TILES_H,),
        in_specs=[
            pl.BlockSpec((TBLK, H), lambda i: (i, 0)),
            pl.BlockSpec((TBLK,), lambda i: (i + NUM_TILES_H,)),
            pl.BlockSpec((NSEG, H), _FULL),
            pl.BlockSpec((NSEG, H), _FULL),
            pl.BlockSpec((3 * H, H), _FULL),
            pl.BlockSpec((H, 1), _FULL),
            pl.BlockSpec((NSEG, H), _FULL),
            pl.BlockSpec((NSEG, 1), _FULL),
            pl.BlockSpec((NSEG, H), _FULL),
        ],
        out_specs=pl.BlockSpec((NSEG, 3 * H), _FULL),
        out_shape=jax.ShapeDtypeStruct((NSEG, 3 * H), jnp.float32),
        scratch_shapes=[
            pltpu.VMEM((NSEG, H), jnp.float32),
            pltpu.VMEM((NSEG, H), jnp.float32),
            pltpu.VMEM((NSEG, 1), jnp.float32),
        ],
    )(em1, seg_col, ss, rr, W, v2, num_p, den_p, bias_p)


def kernel(flat_idx, segment_ids, s_idx, r_idx, ent_embeds, rel_embeds,
           W, b, v_s):
    flat_idx = flat_idx.astype(jnp.int32)
    s_idx = s_idx.astype(jnp.int32)
    r_idx = r_idx.astype(jnp.int32)

    em0, ss, rr = _sc_gather_first(flat_idx, s_idx, r_idx,
                                   ent_embeds, rel_embeds)
    em1 = _sc_gather_second(flat_idx, ent_embeds)

    seg_ids = segment_ids.astype(jnp.int32)
    b2 = b.reshape(1, H)

    num_p, den_p, bias_p = _tc_first(em0, seg_ids, ss, rr, W, b2, v_s)
    row = _tc_second(em1, seg_ids, ss, rr, W, v_s, num_p, den_p, bias_p)
    return row.reshape(B, SEQ_LEN, 3 * H)


# speedup vs baseline: 1.4262x; 1.0332x over previous
import functools

import jax
import jax.numpy as jnp
from jax import lax
from jax.experimental import pallas as pl
from jax.experimental.pallas import tpu as pltpu
from jax.experimental.pallas import tpu_sc as plsc

H = 512
SEQ_LEN = 10
B = 16
NSEG = B * SEQ_LEN
T = 16384
TBLK = 2048
N_HALF = 2
TH = T // N_HALF
NUM_TILES_H = TH // TBLK

SC_NC = 2
SC_NS = 16
SC_NW = SC_NC * SC_NS
ROWS_PER_W = TH // SC_NW
CHUNK = 64
NCH = ROWS_PER_W // CHUNK
SEG_PER_W = 8

def _sc_mesh():
    return plsc.VectorSubcoreMesh(core_axis_name="c", subcore_axis_name="s")


def _em_gather_chunks(idx_h, ent_h, em_o, idx_v, rows0_v, rows1_v,
                      gsem0, gsem1, ssem0, ssem1, idx_off, wid):
    base_w = wid * ROWS_PER_W
    pltpu.sync_copy(idx_h.at[pl.ds(idx_off + base_w, ROWS_PER_W)], idx_v)
    bufs = (rows0_v, rows1_v)
    gsems = (gsem0, gsem1)
    ssems = (ssem0, ssem1)

    def start_gather(c):
        b = c % 2
        return pltpu.async_copy(
            ent_h.at[idx_v.at[pl.ds(c * CHUNK, CHUNK)]], bufs[b], gsems[b])

    def start_store(c):
        b = c % 2
        return pltpu.async_copy(
            bufs[b], em_o.at[pl.ds(base_w + c * CHUNK, CHUNK)], ssems[b])

    hg = [None] * NCH
    hs = [None] * NCH
    hg[0] = start_gather(0)
    for c in range(NCH):
        if c + 1 < NCH:
            if c >= 1:
                hs[c - 1].wait()
            hg[c + 1] = start_gather(c + 1)
        hg[c].wait()
        hs[c] = start_store(c)
    hs[NCH - 2].wait()
    hs[NCH - 1].wait()


def _sc_gather_first(flat_idx, s_idx, r_idx, ent_embeds, rel_embeds):
    @functools.partial(
        pl.kernel,
        mesh=_sc_mesh(),
        out_type=(
            jax.ShapeDtypeStruct((TH, H), jnp.float32),
            jax.ShapeDtypeStruct((NSEG, H), jnp.float32),
            jax.ShapeDtypeStruct((NSEG, H), jnp.float32),
        ),
        scratch_types=[
            pltpu.VMEM((ROWS_PER_W,), jnp.int32),
            pltpu.VMEM((CHUNK, H), jnp.float32),
            pltpu.VMEM((CHUNK, H), jnp.float32),
            pltpu.VMEM((SEG_PER_W,), jnp.int32),
            pltpu.VMEM((SEG_PER_W,), jnp.int32),
            pltpu.VMEM((SEG_PER_W, H), jnp.float32),
            pltpu.VMEM((SEG_PER_W, H), jnp.float32),
            pltpu.SemaphoreType.DMA,
            pltpu.SemaphoreType.DMA,
            pltpu.SemaphoreType.DMA,
            pltpu.SemaphoreType.DMA,
            pltpu.SemaphoreType.DMA,
            pltpu.SemaphoreType.DMA,
        ],
    )
    def gather_k(idx_h, sidx_h, ridx_h, ent_h, rel_h, em_o, ss_o, rr_o,
                 idx_v, rows0_v, rows1_v, idx_s, idx_r, rows_s, rows_r,
                 gsem0, gsem1, ssem0, ssem1, sem_s, sem_r):
        wid = lax.axis_index("s") * SC_NC + lax.axis_index("c")
        is_seg = wid < NSEG // SEG_PER_W
        sb = wid * SEG_PER_W

        @pl.when(is_seg)
        def _seg_start():
            pltpu.sync_copy(sidx_h.at[pl.ds(sb, SEG_PER_W)], idx_s)
            pltpu.sync_copy(ridx_h.at[pl.ds(sb, SEG_PER_W)], idx_r)
            pltpu.async_copy(ent_h.at[idx_s], rows_s, sem_s)
            pltpu.async_copy(rel_h.at[idx_r], rows_r, sem_r)

        _em_gather_chunks(idx_h, ent_h, em_o, idx_v, rows0_v, rows1_v,
                          gsem0, gsem1, ssem0, ssem1, 0, wid)

        @pl.when(is_seg)
        def _seg_drain():
            pltpu.make_async_copy(ent_h.at[idx_s], rows_s, sem_s).wait()
            pltpu.sync_copy(rows_s, ss_o.at[pl.ds(sb, SEG_PER_W)])
            pltpu.make_async_copy(rel_h.at[idx_r], rows_r, sem_r).wait()
            pltpu.sync_copy(rows_r, rr_o.at[pl.ds(sb, SEG_PER_W)])

    return gather_k(flat_idx, s_idx, r_idx, ent_embeds, rel_embeds)


def _sc_gather_second(flat_idx, ent_embeds):
    @functools.partial(
        pl.kernel,
        mesh=_sc_mesh(),
        out_type=jax.ShapeDtypeStruct((TH, H), jnp.float32),
        scratch_types=[
            pltpu.VMEM((ROWS_PER_W,), jnp.int32),
            pltpu.VMEM((CHUNK, H), jnp.float32),
            pltpu.VMEM((CHUNK, H), jnp.float32),
            pltpu.SemaphoreType.DMA,
            pltpu.SemaphoreType.DMA,
            pltpu.SemaphoreType.DMA,
            pltpu.SemaphoreType.DMA,
        ],
    )
    def gather_k(idx_h, ent_h, em_o, idx_v, rows0_v, rows1_v,
                 gsem0, gsem1, ssem0, ssem1):
        wid = lax.axis_index("s") * SC_NC + lax.axis_index("c")
        _em_gather_chunks(idx_h, ent_h, em_o, idx_v, rows0_v, rows1_v,
                          gsem0, gsem1, ssem0, ssem1, TH, wid)

    return gather_k(flat_idx, ent_embeds)


def _attn_tile(em_ref, seg_ref, w_ref, v_ref, bias_s, num_s, den_s):
    em = em_ref[...]
    seg = seg_ref[...]
    onehot_t = (seg[None, :] == lax.broadcasted_iota(
        jnp.int32, (NSEG, TBLK), 0)).astype(jnp.float32)
    ohb = onehot_t.astype(jnp.bfloat16)
    bias_t = lax.dot_general(bias_s[...].astype(jnp.bfloat16), ohb,
                             (((0,), (0,)), ((), ())),
                             preferred_element_type=jnp.float32)
    zt = lax.dot_general(w_ref[0:H, :].astype(jnp.bfloat16),
                         em.astype(jnp.bfloat16),
                         (((0,), (1,)), ((), ())),
                         preferred_element_type=jnp.float32) + bias_t
    zat = jnp.tanh(zt)
    v = v_ref[...]
    s_row = jnp.sum(zat * v, axis=0, keepdims=True)
    m_bound = jnp.sum(jnp.abs(v))
    e_row = jnp.exp(s_row - m_bound)
    ohe = onehot_t * e_row
    num_s[...] += lax.dot_general(ohe, em, (((1,), (0,)), ((), ())),
                                  preferred_element_type=jnp.float32)
    den_s[...] += jnp.sum(ohe, axis=1, keepdims=True)


def _attn_body_first(em_ref, seg_ref, ss_ref, rr_ref, w_ref, b_ref, v_ref,
                     num_o, den_o, bias_o, bias_s, num_s, den_s):
    i = pl.program_id(0)

    @pl.when(i == 0)
    def _init():
        bias_s[...] = (
            jnp.dot(ss_ref[...].astype(jnp.bfloat16),
                    w_ref[H:2 * H, :].astype(jnp.bfloat16),
                    preferred_element_type=jnp.float32)
            + jnp.dot(rr_ref[...].astype(jnp.bfloat16),
                      w_ref[2 * H:3 * H, :].astype(jnp.bfloat16),
                      preferred_element_type=jnp.float32)
            + b_ref[...])
        num_s[...] = jnp.zeros_like(num_s)
        den_s[...] = jnp.zeros_like(den_s)

    _attn_tile(em_ref, seg_ref, w_ref, v_ref, bias_s, num_s, den_s)

    @pl.when(i == pl.num_programs(0) - 1)
    def _fin():
        num_o[...] = num_s[...]
        den_o[...] = den_s[...]
        bias_o[...] = bias_s[...]


def _attn_body_second(em_ref, seg_ref, ss_ref, rr_ref, w_ref, v_ref,
                      num_in, den_in, bias_in,
                      row_o, bias_s, num_s, den_s):
    i = pl.program_id(0)

    @pl.when(i == 0)
    def _init():
        bias_s[...] = bias_in[...]
        num_s[...] = num_in[...]
        den_s[...] = den_in[...]

    _attn_tile(em_ref, seg_ref, w_ref, v_ref, bias_s, num_s, den_s)

    @pl.when(i == pl.num_programs(0) - 1)
    def _fin():
        den = den_s[...]
        mask = (den > 0).astype(jnp.float32)
        agg = num_s[...] / jnp.maximum(den, 1e-37)
        row_o[:, 0:H] = agg * mask
        row_o[:, H:2 * H] = ss_ref[...] * mask
        row_o[:, 2 * H:3 * H] = rr_ref[...] * mask


_FULL = lambda i: (0, 0)


def _tc_first(em0, seg_col, ss, rr, W, b2, v2):
    return pl.pallas_call(
        _attn_body_first,
        grid=(NUM_TILES_H,),
        in_specs=[
            pl.BlockSpec((TBLK, H), lambda i: (i, 0)),
            pl.BlockSpec((TBLK,), lambda i: (i,)),
            pl.BlockSpec((NSEG, H), _FULL),
            pl.BlockSpec((NSEG, H), _FULL),
            pl.BlockSpec((3 * H, H), _FULL),
            pl.BlockSpec((1, H), _FULL),
            pl.BlockSpec((H, 1), _FULL),
        ],
        out_specs=[
            pl.BlockSpec((NSEG, H), _FULL),
            pl.BlockSpec((NSEG, 1), _FULL),
            pl.BlockSpec((NSEG, H), _FULL),
        ],
        out_shape=[
            jax.ShapeDtypeStruct((NSEG, H), jnp.float32),
            jax.ShapeDtypeStruct((NSEG, 1), jnp.float32),
            jax.ShapeDtypeStruct((NSEG, H), jnp.float32),
        ],
        scratch_shapes=[
            pltpu.VMEM((NSEG, H), jnp.float32),
            pltpu.VMEM((NSEG, H), jnp.float32),
            pltpu.VMEM((NSEG, 1), jnp.float32),
        ],
    )(em0, seg_col, ss, rr, W, b2, v2)


def _tc_second(em1, seg_col, ss, rr, W, v2, num_p, den_p, bias_p):
    return pl.pallas_call(
        _attn_body_second,
        grid=(NUM_TILES_H,),
        in_specs=[
            pl.BlockSpec((TBLK, H), lambda i: (i, 0)),
            pl.BlockSpec((TBLK,), lambda i: (i + NUM_TILES_H,)),
            pl.BlockSpec((NSEG, H), _FULL),
            pl.BlockSpec((NSEG, H), _FULL),
            pl.BlockSpec((3 * H, H), _FULL),
            pl.BlockSpec((H, 1), _FULL),
            pl.BlockSpec((NSEG, H), _FULL),
            pl.BlockSpec((NSEG, 1), _FULL),
            pl.BlockSpec((NSEG, H), _FULL),
        ],
        out_specs=pl.BlockSpec((NSEG, 3 * H), _FULL),
        out_shape=jax.ShapeDtypeStruct((NSEG, 3 * H), jnp.float32),
        scratch_shapes=[
            pltpu.VMEM((NSEG, H), jnp.float32),
            pltpu.VMEM((NSEG, H), jnp.float32),
            pltpu.VMEM((NSEG, 1), jnp.float32),
        ],
    )(em1, seg_col, ss, rr, W, v2, num_p, den_p, bias_p)


def kernel(flat_idx, segment_ids, s_idx, r_idx, ent_embeds, rel_embeds,
           W, b, v_s):
    flat_idx = flat_idx.astype(jnp.int32)
    s_idx = s_idx.astype(jnp.int32)
    r_idx = r_idx.astype(jnp.int32)

    em0, ss, rr = _sc_gather_first(flat_idx, s_idx, r_idx,
                                   ent_embeds, rel_embeds)
    em1 = _sc_gather_second(flat_idx, ent_embeds)

    seg_ids = segment_ids.astype(jnp.int32)
    b2 = b.reshape(1, H)

    num_p, den_p, bias_p = _tc_first(em0, seg_ids, ss, rr, W, b2, v_s)
    row = _tc_second(em1, seg_ids, ss, rr, W, v_s, num_p, den_p, bias_p)
    return row.reshape(B, SEQ_LEN, 3 * H)


# speedup vs baseline: 1.4407x; 1.0102x over previous
import functools

import jax
import jax.numpy as jnp
from jax import lax
from jax.experimental import pallas as pl
from jax.experimental.pallas import tpu as pltpu
from jax.experimental.pallas import tpu_sc as plsc

H = 512
SEQ_LEN = 10
B = 16
NSEG = B * SEQ_LEN
T = 16384
TBLK = 4096
N_HALF = 2
TH = T // N_HALF
NUM_TILES_H = TH // TBLK

SC_NC = 2
SC_NS = 16
SC_NW = SC_NC * SC_NS
ROWS_PER_W = TH // SC_NW
CHUNK = 64
NCH = ROWS_PER_W // CHUNK
SEG_PER_W = 8

def _sc_mesh():
    return plsc.VectorSubcoreMesh(core_axis_name="c", subcore_axis_name="s")


def _em_gather_chunks(idx_h, ent_h, em_o, idx_v, rows0_v, rows1_v,
                      gsem0, gsem1, ssem0, ssem1, idx_off, wid):
    base_w = wid * ROWS_PER_W
    pltpu.sync_copy(idx_h.at[pl.ds(idx_off + base_w, ROWS_PER_W)], idx_v)
    bufs = (rows0_v, rows1_v)
    gsems = (gsem0, gsem1)
    ssems = (ssem0, ssem1)

    def start_gather(c):
        b = c % 2
        return pltpu.async_copy(
            ent_h.at[idx_v.at[pl.ds(c * CHUNK, CHUNK)]], bufs[b], gsems[b])

    def start_store(c):
        b = c % 2
        return pltpu.async_copy(
            bufs[b], em_o.at[pl.ds(base_w + c * CHUNK, CHUNK)], ssems[b])

    hg = [None] * NCH
    hs = [None] * NCH
    hg[0] = start_gather(0)
    for c in range(NCH):
        if c + 1 < NCH:
            if c >= 1:
                hs[c - 1].wait()
            hg[c + 1] = start_gather(c + 1)
        hg[c].wait()
        hs[c] = start_store(c)
    hs[NCH - 2].wait()
    hs[NCH - 1].wait()


def _sc_gather_first(flat_idx, s_idx, r_idx, ent_embeds, rel_embeds):
    @functools.partial(
        pl.kernel,
        mesh=_sc_mesh(),
        out_type=(
            jax.ShapeDtypeStruct((TH, H), jnp.float32),
            jax.ShapeDtypeStruct((NSEG, H), jnp.float32),
            jax.ShapeDtypeStruct((NSEG, H), jnp.float32),
        ),
        scratch_types=[
            pltpu.VMEM((ROWS_PER_W,), jnp.int32),
            pltpu.VMEM((CHUNK, H), jnp.float32),
            pltpu.VMEM((CHUNK, H), jnp.float32),
            pltpu.VMEM((SEG_PER_W,), jnp.int32),
            pltpu.VMEM((SEG_PER_W,), jnp.int32),
            pltpu.VMEM((SEG_PER_W, H), jnp.float32),
            pltpu.VMEM((SEG_PER_W, H), jnp.float32),
            pltpu.SemaphoreType.DMA,
            pltpu.SemaphoreType.DMA,
            pltpu.SemaphoreType.DMA,
            pltpu.SemaphoreType.DMA,
            pltpu.SemaphoreType.DMA,
            pltpu.SemaphoreType.DMA,
        ],
    )
    def gather_k(idx_h, sidx_h, ridx_h, ent_h, rel_h, em_o, ss_o, rr_o,
                 idx_v, rows0_v, rows1_v, idx_s, idx_r, rows_s, rows_r,
                 gsem0, gsem1, ssem0, ssem1, sem_s, sem_r):
        wid = lax.axis_index("s") * SC_NC + lax.axis_index("c")
        is_seg = wid < NSEG // SEG_PER_W
        sb = wid * SEG_PER_W

        @pl.when(is_seg)
        def _seg_start():
            pltpu.sync_copy(sidx_h.at[pl.ds(sb, SEG_PER_W)], idx_s)
            pltpu.sync_copy(ridx_h.at[pl.ds(sb, SEG_PER_W)], idx_r)
            pltpu.async_copy(ent_h.at[idx_s], rows_s, sem_s)
            pltpu.async_copy(rel_h.at[idx_r], rows_r, sem_r)

        _em_gather_chunks(idx_h, ent_h, em_o, idx_v, rows0_v, rows1_v,
                          gsem0, gsem1, ssem0, ssem1, 0, wid)

        @pl.when(is_seg)
        def _seg_drain():
            pltpu.make_async_copy(ent_h.at[idx_s], rows_s, sem_s).wait()
            pltpu.sync_copy(rows_s, ss_o.at[pl.ds(sb, SEG_PER_W)])
            pltpu.make_async_copy(rel_h.at[idx_r], rows_r, sem_r).wait()
            pltpu.sync_copy(rows_r, rr_o.at[pl.ds(sb, SEG_PER_W)])

    return gather_k(flat_idx, s_idx, r_idx, ent_embeds, rel_embeds)


def _sc_gather_second(flat_idx, ent_embeds):
    @functools.partial(
        pl.kernel,
        mesh=_sc_mesh(),
        out_type=jax.ShapeDtypeStruct((TH, H), jnp.float32),
        scratch_types=[
            pltpu.VMEM((ROWS_PER_W,), jnp.int32),
            pltpu.VMEM((CHUNK, H), jnp.float32),
            pltpu.VMEM((CHUNK, H), jnp.float32),
            pltpu.SemaphoreType.DMA,
            pltpu.SemaphoreType.DMA,
            pltpu.SemaphoreType.DMA,
            pltpu.SemaphoreType.DMA,
        ],
    )
    def gather_k(idx_h, ent_h, em_o, idx_v, rows0_v, rows1_v,
                 gsem0, gsem1, ssem0, ssem1):
        wid = lax.axis_index("s") * SC_NC + lax.axis_index("c")
        _em_gather_chunks(idx_h, ent_h, em_o, idx_v, rows0_v, rows1_v,
                          gsem0, gsem1, ssem0, ssem1, TH, wid)

    return gather_k(flat_idx, ent_embeds)


def _attn_tile(em_ref, seg_ref, w_ref, v_ref, bias_s, num_s, den_s):
    em = em_ref[...]
    seg = seg_ref[...]
    onehot_t = (seg[None, :] == lax.broadcasted_iota(
        jnp.int32, (NSEG, TBLK), 0)).astype(jnp.float32)
    ohb = onehot_t.astype(jnp.bfloat16)
    bias_t = lax.dot_general(bias_s[...].astype(jnp.bfloat16), ohb,
                             (((0,), (0,)), ((), ())),
                             preferred_element_type=jnp.float32)
    zt = lax.dot_general(w_ref[0:H, :].astype(jnp.bfloat16),
                         em.astype(jnp.bfloat16),
                         (((0,), (1,)), ((), ())),
                         preferred_element_type=jnp.float32) + bias_t
    zat = jnp.tanh(zt)
    v = v_ref[...]
    s_row = jnp.sum(zat * v, axis=0, keepdims=True)
    m_bound = jnp.sum(jnp.abs(v))
    e_row = jnp.exp(s_row - m_bound)
    ohe = onehot_t * e_row
    num_s[...] += lax.dot_general(ohe, em, (((1,), (0,)), ((), ())),
                                  preferred_element_type=jnp.float32)
    den_s[...] += jnp.sum(ohe, axis=1, keepdims=True)


def _attn_body_first(em_ref, seg_ref, ss_ref, rr_ref, w_ref, b_ref, v_ref,
                     num_o, den_o, bias_o, bias_s, num_s, den_s):
    i = pl.program_id(0)

    @pl.when(i == 0)
    def _init():
        bias_s[...] = (
            jnp.dot(ss_ref[...].astype(jnp.bfloat16),
                    w_ref[H:2 * H, :].astype(jnp.bfloat16),
                    preferred_element_type=jnp.float32)
            + jnp.dot(rr_ref[...].astype(jnp.bfloat16),
                      w_ref[2 * H:3 * H, :].astype(jnp.bfloat16),
                      preferred_element_type=jnp.float32)
            + b_ref[...])
        num_s[...] = jnp.zeros_like(num_s)
        den_s[...] = jnp.zeros_like(den_s)

    _attn_tile(em_ref, seg_ref, w_ref, v_ref, bias_s, num_s, den_s)

    @pl.when(i == pl.num_programs(0) - 1)
    def _fin():
        num_o[...] = num_s[...]
        den_o[...] = den_s[...]
        bias_o[...] = bias_s[...]


def _attn_body_second(em_ref, seg_ref, ss_ref, rr_ref, w_ref, v_ref,
                      num_in, den_in, bias_in,
                      row_o, bias_s, num_s, den_s):
    i = pl.program_id(0)

    @pl.when(i == 0)
    def _init():
        bias_s[...] = bias_in[...]
        num_s[...] = num_in[...]
        den_s[...] = den_in[...]

    _attn_tile(em_ref, seg_ref, w_ref, v_ref, bias_s, num_s, den_s)

    @pl.when(i == pl.num_programs(0) - 1)
    def _fin():
        den = den_s[...]
        mask = (den > 0).astype(jnp.float32)
        agg = num_s[...] / jnp.maximum(den, 1e-37)
        row_o[:, 0:H] = agg * mask
        row_o[:, H:2 * H] = ss_ref[...] * mask
        row_o[:, 2 * H:3 * H] = rr_ref[...] * mask


_FULL = lambda i: (0, 0)


def _tc_first(em0, seg_col, ss, rr, W, b2, v2):
    return pl.pallas_call(
        _attn_body_first,
        grid=(NUM_TILES_H,),
        in_specs=[
            pl.BlockSpec((TBLK, H), lambda i: (i, 0)),
            pl.BlockSpec((TBLK,), lambda i: (i,)),
            pl.BlockSpec((NSEG, H), _FULL),
            pl.BlockSpec((NSEG, H), _FULL),
            pl.BlockSpec((3 * H, H), _FULL),
            pl.BlockSpec((1, H), _FULL),
            pl.BlockSpec((H, 1), _FULL),
        ],
        out_specs=[
            pl.BlockSpec((NSEG, H), _FULL),
            pl.BlockSpec((NSEG, 1), _FULL),
            pl.BlockSpec((NSEG, H), _FULL),
        ],
        out_shape=[
            jax.ShapeDtypeStruct((NSEG, H), jnp.float32),
            jax.ShapeDtypeStruct((NSEG, 1), jnp.float32),
            jax.ShapeDtypeStruct((NSEG, H), jnp.float32),
        ],
        scratch_shapes=[
            pltpu.VMEM((NSEG, H), jnp.float32),
            pltpu.VMEM((NSEG, H), jnp.float32),
            pltpu.VMEM((NSEG, 1), jnp.float32),
        ],
    )(em0, seg_col, ss, rr, W, b2, v2)


def _tc_second(em1, seg_col, ss, rr, W, v2, num_p, den_p, bias_p):
    return pl.pallas_call(
        _attn_body_second,
        grid=(NUM_TILES_H,),
        in_specs=[
            pl.BlockSpec((TBLK, H), lambda i: (i, 0)),
            pl.BlockSpec((TBLK,), lambda i: (i + NUM_TILES_H,)),
            pl.BlockSpec((NSEG, H), _FULL),
            pl.BlockSpec((NSEG, H), _FULL),
            pl.BlockSpec((3 * H, H), _FULL),
            pl.BlockSpec((H, 1), _FULL),
            pl.BlockSpec((NSEG, H), _FULL),
            pl.BlockSpec((NSEG, 1), _FULL),
            pl.BlockSpec((NSEG, H), _FULL),
        ],
        out_specs=pl.BlockSpec((NSEG, 3 * H), _FULL),
        out_shape=jax.ShapeDtypeStruct((NSEG, 3 * H), jnp.float32),
        scratch_shapes=[
            pltpu.VMEM((NSEG, H), jnp.float32),
            pltpu.VMEM((NSEG, H), jnp.float32),
            pltpu.VMEM((NSEG, 1), jnp.float32),
        ],
    )(em1, seg_col, ss, rr, W, v2, num_p, den_p, bias_p)


def kernel(flat_idx, segment_ids, s_idx, r_idx, ent_embeds, rel_embeds,
           W, b, v_s):
    flat_idx = flat_idx.astype(jnp.int32)
    s_idx = s_idx.astype(jnp.int32)
    r_idx = r_idx.astype(jnp.int32)

    em0, ss, rr = _sc_gather_first(flat_idx, s_idx, r_idx,
                                   ent_embeds, rel_embeds)
    em1 = _sc_gather_second(flat_idx, ent_embeds)

    seg_ids = segment_ids.astype(jnp.int32)
    b2 = b.reshape(1, H)

    num_p, den_p, bias_p = _tc_first(em0, seg_ids, ss, rr, W, b2, v_s)
    row = _tc_second(em1, seg_ids, ss, rr, W, v_s, num_p, den_p, bias_p)
    return row.reshape(B, SEQ_LEN, 3 * H)
